# baseline (speedup 1.0000x reference)
import jax
import jax.numpy as jnp
from jax import lax
from jax.experimental import pallas as pl
from jax.experimental.pallas import tpu as pltpu
from jax.experimental.pallas import tpu_sc as plsc

N = 5000
NPAD = 5120
B = 160
NB = NPAD // B
NS = 16
L = 16
VPB = B // L
HV = VPB // 2
NMS_THRESH = 0.75


def _nms_body(x1h, y1h, x2h, y2h, keep_out,
              x1v, y1v, x2v, y2v, areav, keepv, pub):
    wid = lax.axis_index("s")

    pltpu.sync_copy(x1h, x1v)
    pltpu.sync_copy(y1h, y1v)
    pltpu.sync_copy(x2h, x2v)
    pltpu.sync_copy(y2h, y2v)

    def _init(i, _):
        sl = pl.ds(i * L, L)
        areav[sl] = (x2v[sl] - x1v[sl]) * (y2v[sl] - y1v[sl])
        keepv[sl] = jnp.full((L,), 1.0, jnp.float32)
        return 0
    lax.fori_loop(0, NPAD // L, _init, 0)

    def _load_pivots(pvbase):
        sl = pl.ds(pvbase, L)
        return (x1v[sl], y1v[sl], x2v[sl], y2v[sl], areav[sl])

    def _sup_factor(px1, py1, px2, py2, tpa, omk,
                    cx1, cy1, cx2, cy2, tca):
        iw = jnp.maximum(jnp.minimum(px2, cx2) - jnp.maximum(px1, cx1), 0.0)
        ih = jnp.maximum(jnp.minimum(py2, cy2) - jnp.maximum(py1, cy1), 0.0)
        inter = iw * ih
        sup = (1.0 + NMS_THRESH) * inter > tpa + tca
        return jnp.where(sup, omk, 1.0)

    def _cross_update(pbase, cbase):
        for half in range(2):
            hbase = cbase + half * (HV * L)
            sls = [pl.ds(hbase + k * L, L) for k in range(HV)]
            cx1 = [x1v[s] for s in sls]
            cy1 = [y1v[s] for s in sls]
            cx2 = [x2v[s] for s in sls]
            cy2 = [y2v[s] for s in sls]
            tca = [NMS_THRESH * areav[s] for s in sls]
            kc0 = tuple(keepv[s] for s in sls)

            def _pg(g, kcs):
                pvbase = pbase + g * L
                px1v, py1v, px2v, py2v, pav = _load_pivots(pvbase)
                kgv = keepv[pl.ds(pvbase, L)]
                tpav = NMS_THRESH * pav
                kcs = list(kcs)
                for i in range(L):
                    omk = 1.0 - kgv[i]
                    px1, py1, px2, py2, tpa = (px1v[i], py1v[i], px2v[i],
                                               py2v[i], tpav[i])
                    for k in range(HV):
                        f = _sup_factor(px1, py1, px2, py2, tpa, omk,
                                        cx1[k], cy1[k], cx2[k], cy2[k],
                                        tca[k])
                        kcs[k] = kcs[k] * f
                return tuple(kcs)

            kcs = lax.fori_loop(0, VPB, _pg, kc0)
            for k in range(HV):
                keepv[sls[k]] = kcs[k]

    def _inblock(cbase):
        lane = lax.iota(jnp.int32, L)

        def _pg(g, _):
            pvbase = cbase + g * L
            px1v, py1v, px2v, py2v, pav = _load_pivots(pvbase)
            kgv = keepv[pl.ds(pvbase, L)]
            tpav = NMS_THRESH * pav
            for i in range(L):
                omk = 1.0 - kgv[i]
                px1, py1, px2, py2, tpa = (px1v[i], py1v[i], px2v[i],
                                           py2v[i], tpav[i])
                iw = jnp.maximum(
                    jnp.minimum(px2, px2v) - jnp.maximum(px1, px1v), 0.0)
                ih = jnp.maximum(
                    jnp.minimum(py2, py2v) - jnp.maximum(py1, py1v), 0.0)
                inter = iw * ih
                sup = (1.0 + NMS_THRESH) * inter > tpa + tpav
                kg_sup = jnp.where(lane > i, kgv * omk, kgv)
                kgv = jnp.where(sup, kg_sup, kgv)
            keepv[pl.ds(pvbase, L)] = kgv

            def _dv(v, _):
                sl = pl.ds(cbase + v * L, L)
                cx1, cy1, cx2, cy2 = x1v[sl], y1v[sl], x2v[sl], y2v[sl]
                tca = NMS_THRESH * areav[sl]
                facs = [_sup_factor(px1v[i], py1v[i], px2v[i], py2v[i],
                                    tpav[i], 1.0 - kgv[i],
                                    cx1, cy1, cx2, cy2, tca)
                        for i in range(L)]
                while len(facs) > 1:
                    facs = [facs[j] * facs[j + 1]
                            for j in range(0, len(facs), 2)]
                keepv[sl] = keepv[sl] * facs[0]
                return 0
            lax.fori_loop(g + 1, VPB, _dv, 0)
            return 0
        lax.fori_loop(0, VPB, _pg, 0)

    def _round(c, _):
        prev = c - 1

        @pl.when(c > 0)
        def _():
            psl = pl.ds(prev * B, B)
            pltpu.sync_copy(pub.at[psl], keepv.at[psl])

            def _own(k, _):
                ob = jnp.where(k == 0, wid, (NB - 1) - wid)

                @pl.when(ob >= c)
                def _():
                    _cross_update(prev * B, ob * B)
                return 0
            lax.fori_loop(0, 2, _own, 0)

        @pl.when(jnp.minimum(c, (NB - 1) - c) == wid)
        def _():
            cbase = c * B
            _inblock(cbase)
            csl = pl.ds(cbase, B)
            pltpu.sync_copy(keepv.at[csl], pub.at[csl])

        plsc.subcore_barrier()
        return 0

    lax.fori_loop(0, NB, _round, 0)

    for ob in (wid, (NB - 1) - wid):
        osl = pl.ds(ob * B, B)
        pltpu.sync_copy(keepv.at[osl], keep_out.at[osl])


@jax.jit
def _nms_keep(x1, y1, x2, y2):
    mesh = plsc.VectorSubcoreMesh(
        core_axis_name="c", subcore_axis_name="s", num_cores=1)
    f = pl.kernel(
        _nms_body,
        out_type=jax.ShapeDtypeStruct((NPAD,), jnp.float32),
        mesh=mesh,
        scratch_types=[
            pltpu.VMEM((NPAD,), jnp.float32),
            pltpu.VMEM((NPAD,), jnp.float32),
            pltpu.VMEM((NPAD,), jnp.float32),
            pltpu.VMEM((NPAD,), jnp.float32),
            pltpu.VMEM((NPAD,), jnp.float32),
            pltpu.VMEM((NPAD,), jnp.float32),
            pltpu.VMEM_SHARED((NPAD,), jnp.float32),
        ],
    )
    return f(x1, y1, x2, y2)


def kernel(boxes, scores, classes):
    max_coord = jnp.max(boxes) + 1.0
    offsets = classes.astype(boxes.dtype) * max_coord
    boxes_off = boxes + offsets[:, None]
    order = jnp.argsort(-scores)
    b_sorted = jnp.take(boxes_off, order, axis=0)
    b_orig_sorted = jnp.take(boxes, order, axis=0)
    s_sorted = jnp.take(scores, order, axis=0)

    pad = jnp.full((NPAD - N,), -1e6, jnp.float32)
    x1 = jnp.concatenate([b_sorted[:, 0], pad])
    y1 = jnp.concatenate([b_sorted[:, 1], pad])
    x2 = jnp.concatenate([b_sorted[:, 2], pad])
    y2 = jnp.concatenate([b_sorted[:, 3], pad])

    keepf = _nms_keep(x1, y1, x2, y2)[:N]
    out = jnp.concatenate(
        [b_orig_sorted * keepf[:, None], (s_sorted * keepf)[:, None]], axis=1)
    return out


# speedup vs baseline: 1.2766x; 1.2766x over previous
import jax
import jax.numpy as jnp
from jax import lax
from jax.experimental import pallas as pl
from jax.experimental.pallas import tpu as pltpu
from jax.experimental.pallas import tpu_sc as plsc

N = 5000
NPAD = 5120
B = 160
NB = NPAD // B
NS = 16
L = 16
VPB = B // L
HV = VPB // 2
NMS_THRESH = 0.75


def _nms_body(x1h, y1h, x2h, y2h, keep_out,
              x1v, y1v, x2v, y2v, areav, keepv, pub):
    wid = lax.axis_index("s")

    pltpu.sync_copy(x1h, x1v)
    pltpu.sync_copy(y1h, y1v)
    pltpu.sync_copy(x2h, x2v)
    pltpu.sync_copy(y2h, y2v)

    def _init(i, _):
        sl = pl.ds(i * L, L)
        areav[sl] = (x2v[sl] - x1v[sl]) * (y2v[sl] - y1v[sl])
        keepv[sl] = jnp.full((L,), 1.0, jnp.float32)
        return 0
    lax.fori_loop(0, NPAD // L, _init, 0)

    def _load_pivots(pvbase):
        sl = pl.ds(pvbase, L)
        return (x1v[sl], y1v[sl], x2v[sl], y2v[sl], areav[sl])

    def _pair_update(px1, py1, px2, py2, pa, omk,
                     cx1, cy1, cx2, cy2, ca, kc):
        iw = jnp.maximum(jnp.minimum(px2, cx2) - jnp.maximum(px1, cx1), 0.0)
        ih = jnp.maximum(jnp.minimum(py2, cy2) - jnp.maximum(py1, cy1), 0.0)
        inter = iw * ih
        sup = inter > NMS_THRESH * (pa + ca - inter)
        return jnp.where(sup, kc * omk, kc)

    def _cross_update(pbase, cbase):
        for half in range(2):
            hbase = cbase + half * (HV * L)
            sls = [pl.ds(hbase + k * L, L) for k in range(HV)]
            cx1 = [x1v[s] for s in sls]
            cy1 = [y1v[s] for s in sls]
            cx2 = [x2v[s] for s in sls]
            cy2 = [y2v[s] for s in sls]
            ca = [areav[s] for s in sls]
            kc0 = tuple(keepv[s] for s in sls)

            def _pg(g, kcs):
                pvbase = pbase + g * L
                px1v, py1v, px2v, py2v, pav = _load_pivots(pvbase)
                kgv = keepv[pl.ds(pvbase, L)]
                kcs = list(kcs)
                for i in range(L):
                    omk = 1.0 - kgv[i]
                    px1, py1, px2, py2, pa = (px1v[i], py1v[i], px2v[i],
                                              py2v[i], pav[i])
                    for k in range(HV):
                        kcs[k] = _pair_update(px1, py1, px2, py2, pa, omk,
                                              cx1[k], cy1[k], cx2[k], cy2[k],
                                              ca[k], kcs[k])
                return tuple(kcs)

            kcs = lax.fori_loop(0, VPB, _pg, kc0)
            for k in range(HV):
                keepv[sls[k]] = kcs[k]

    def _inblock(cbase):
        lane = lax.iota(jnp.int32, L)

        def _pg(g, _):
            pvbase = cbase + g * L
            px1v, py1v, px2v, py2v, pav = _load_pivots(pvbase)
            kgv = keepv[pl.ds(pvbase, L)]
            for i in range(L):
                omk = 1.0 - kgv[i]
                px1, py1, px2, py2, pa = (px1v[i], py1v[i], px2v[i],
                                          py2v[i], pav[i])
                iw = jnp.maximum(
                    jnp.minimum(px2, px2v) - jnp.maximum(px1, px1v), 0.0)
                ih = jnp.maximum(
                    jnp.minimum(py2, py2v) - jnp.maximum(py1, py1v), 0.0)
                inter = iw * ih
                sup = inter > NMS_THRESH * (pa + pav - inter)
                kg_sup = jnp.where(lane > i, kgv * omk, kgv)
                kgv = jnp.where(sup, kg_sup, kgv)
            keepv[pl.ds(pvbase, L)] = kgv

            def _dv(v, _):
                sl = pl.ds(cbase + v * L, L)
                cx1, cy1, cx2, cy2 = x1v[sl], y1v[sl], x2v[sl], y2v[sl]
                ca = areav[sl]
                kc = keepv[sl]
                for i in range(L):
                    omk = 1.0 - kgv[i]
                    kc = _pair_update(px1v[i], py1v[i], px2v[i], py2v[i],
                                      pav[i], omk, cx1, cy1, cx2, cy2, ca, kc)
                keepv[sl] = kc
                return 0
            lax.fori_loop(g + 1, VPB, _dv, 0)
            return 0
        lax.fori_loop(0, VPB, _pg, 0)

    def _round(c, _):
        prev = c - 1

        @pl.when(c > 0)
        def _():
            psl = pl.ds(prev * B, B)
            pltpu.sync_copy(pub.at[psl], keepv.at[psl])

            ob2 = (NB - 1) - wid
            defer = jnp.logical_and(c == wid,
                                    jnp.logical_and(wid >= 1, wid < NS - 1))
            catchup = jnp.logical_and(c == wid + 1,
                                      jnp.logical_and(wid >= 1, wid < NS - 1))

            @pl.when(wid >= c)
            def _():
                _cross_update(prev * B, wid * B)

            @pl.when(catchup)
            def _():
                _cross_update((c - 2) * B, ob2 * B)

            @pl.when(jnp.logical_and(ob2 >= c, jnp.logical_not(defer)))
            def _():
                _cross_update(prev * B, ob2 * B)

        @pl.when(jnp.minimum(c, (NB - 1) - c) == wid)
        def _():
            cbase = c * B
            _inblock(cbase)
            csl = pl.ds(cbase, B)
            pltpu.sync_copy(keepv.at[csl], pub.at[csl])

        plsc.subcore_barrier()
        return 0

    lax.fori_loop(0, NB, _round, 0)

    for ob in (wid, (NB - 1) - wid):
        osl = pl.ds(ob * B, B)
        pltpu.sync_copy(keepv.at[osl], keep_out.at[osl])


@jax.jit
def _nms_keep(x1, y1, x2, y2):
    mesh = plsc.VectorSubcoreMesh(
        core_axis_name="c", subcore_axis_name="s", num_cores=1)
    f = pl.kernel(
        _nms_body,
        out_type=jax.ShapeDtypeStruct((NPAD,), jnp.float32),
        mesh=mesh,
        scratch_types=[
            pltpu.VMEM((NPAD,), jnp.float32),
            pltpu.VMEM((NPAD,), jnp.float32),
            pltpu.VMEM((NPAD,), jnp.float32),
            pltpu.VMEM((NPAD,), jnp.float32),
            pltpu.VMEM((NPAD,), jnp.float32),
            pltpu.VMEM((NPAD,), jnp.float32),
            pltpu.VMEM_SHARED((NPAD,), jnp.float32),
        ],
    )
    return f(x1, y1, x2, y2)


def kernel(boxes, scores, classes):
    max_coord = jnp.max(boxes) + 1.0
    offsets = classes.astype(boxes.dtype) * max_coord
    boxes_off = boxes + offsets[:, None]
    order = jnp.argsort(-scores)
    b_sorted = jnp.take(boxes_off, order, axis=0)
    b_orig_sorted = jnp.take(boxes, order, axis=0)
    s_sorted = jnp.take(scores, order, axis=0)

    pad = jnp.full((NPAD - N,), -1e6, jnp.float32)
    x1 = jnp.concatenate([b_sorted[:, 0], pad])
    y1 = jnp.concatenate([b_sorted[:, 1], pad])
    x2 = jnp.concatenate([b_sorted[:, 2], pad])
    y2 = jnp.concatenate([b_sorted[:, 3], pad])

    keepf = _nms_keep(x1, y1, x2, y2)[:N]
    out = jnp.concatenate(
        [b_orig_sorted * keepf[:, None], (s_sorted * keepf)[:, None]], axis=1)
    return out


# speedup vs baseline: 3.2392x; 2.5373x over previous
import jax
import jax.numpy as jnp
from jax import lax
from jax.experimental import pallas as pl
from jax.experimental.pallas import tpu as pltpu
from jax.experimental.pallas import tpu_sc as plsc

N = 5000
NPAD = 5120
L = 16
NG = NPAD // L
HV = 5
HB = HV * L
NJ = NPAD // HB
NMS_THRESH = 0.75


def _stage_coords(x1h, y1h, x2h, y2h, x1v, y1v, x2v, y2v, areav):
    pltpu.sync_copy(x1h, x1v)
    pltpu.sync_copy(y1h, y1v)
    pltpu.sync_copy(x2h, x2v)
    pltpu.sync_copy(y2h, y2v)

    def _init(i, _):
        sl = pl.ds(i * L, L)
        areav[sl] = (x2v[sl] - x1v[sl]) * (y2v[sl] - y1v[sl])
        return 0
    lax.fori_loop(0, NG, _init, 0)


def _overlap_body(x1h, y1h, x2h, y2h, sup_out,
                  x1v, y1v, x2v, y2v, areav, stage):
    ci = lax.axis_index("c")
    si = lax.axis_index("s")
    wid = ci * 16 + si

    _stage_coords(x1h, y1h, x2h, y2h, x1v, y1v, x2v, y2v, areav)

    lane = lax.iota(jnp.int32, L)
    onehots = [jnp.where(lane == i, 1.0, 0.0) for i in range(L)]

    def _halfjob(m):
        h0 = m * HV
        base = h0 * L
        sls = [pl.ds(base + k * L, L) for k in range(HV)]
        cx1 = [x1v[s] for s in sls]
        cy1 = [y1v[s] for s in sls]
        cx2 = [x2v[s] for s in sls]
        cy2 = [y2v[s] for s in sls]
        ca = [areav[s] for s in sls]

        def _pg(g, accs):
            psl = pl.ds(g * L, L)
            px1v, py1v, px2v, py2v, pav = (x1v[psl], y1v[psl], x2v[psl],
                                           y2v[psl], areav[psl])
            accs = list(accs)
            def _eq(k):
                d = (g - (h0 + k)).astype(jnp.float32)
                return jnp.maximum(1.0 - d * d, 0.0)
            eqf = [_eq(k) for k in range(HV)]
            for i in range(L):
                px1, py1, px2, py2, pa = (px1v[i], py1v[i], px2v[i],
                                          py2v[i], pav[i])
                for k in range(HV):
                    iw = jnp.maximum(
                        jnp.minimum(px2, cx2[k]) - jnp.maximum(px1, cx1[k]),
                        0.0)
                    ih = jnp.maximum(
                        jnp.minimum(py2, cy2[k]) - jnp.maximum(py1, cy1[k]),
                        0.0)
                    inter = iw * ih
                    sup = inter > NMS_THRESH * (pa + ca[k] - inter)
                    f = jnp.where(sup, 1.0, 0.0) - eqf[k] * onehots[i]
                    accs[k] = jnp.maximum(accs[k], f)
            return tuple(accs)

        zero = jnp.zeros((L,), jnp.float32)
        accs = lax.fori_loop(0, h0 + HV, _pg, (zero,) * HV)
        for k in range(HV):
            stage[pl.ds(k * L, L)] = accs[k]
        pltpu.sync_copy(stage, sup_out.at[pl.ds(base, HB)])

    def _jobs(t, _):
        _halfjob(wid + t * ((NJ - 1) - 2 * wid))
        return 0
    lax.fori_loop(0, 2, _jobs, 0)


def _resolve_body(x1h, y1h, x2h, y2h, suph, keep_out,
                  x1v, y1v, x2v, y2v, areav, keepv, supv):
    si = lax.axis_index("s")

    @pl.when(si == 0)
    def _():
        _stage_coords(x1h, y1h, x2h, y2h, x1v, y1v, x2v, y2v, areav)
        pltpu.sync_copy(suph, supv)

        def _init(i, _):
            keepv[pl.ds(i * L, L)] = jnp.full((L,), 1.0, jnp.float32)
            return 0
        lax.fori_loop(0, NG, _init, 0)

        lane = lax.iota(jnp.int32, L)

        def _vmax(x):
            vals = [x[i] for i in range(L)]
            while len(vals) > 1:
                vals = [jnp.maximum(vals[i], vals[i + 1])
                        for i in range(0, len(vals), 2)]
            return vals[0]

        def _vreg(v, _):
            sl = pl.ds(v * L, L)
            sv = supv[sl]

            @pl.when(_vmax(sv) > 0.0)
            def _():
                for j in range(L):

                    @pl.when(sv[j] > 0.0)
                    def _():
                        jx1 = x1v[sl][j]
                        jy1 = y1v[sl][j]
                        jx2 = x2v[sl][j]
                        jy2 = y2v[sl][j]
                        ja = areav[sl][j]

                        def _pred(u, acc):
                            ps = pl.ds(u * L, L)
                            iw = jnp.maximum(
                                jnp.minimum(jx2, x2v[ps])
                                - jnp.maximum(jx1, x1v[ps]), 0.0)
                            ih = jnp.maximum(
                                jnp.minimum(jy2, y2v[ps])
                                - jnp.maximum(jy1, y1v[ps]), 0.0)
                            inter = iw * ih
                            sup = inter > NMS_THRESH * (ja + areav[ps] - inter)
                            return jnp.maximum(
                                acc, jnp.where(sup, keepv[ps], 0.0))

                        acc = lax.fori_loop(0, v, _pred,
                                            jnp.zeros((L,), jnp.float32))
                        if j > 0:
                            iw = jnp.maximum(
                                jnp.minimum(jx2, x2v[sl])
                                - jnp.maximum(jx1, x1v[sl]), 0.0)
                            ih = jnp.maximum(
                                jnp.minimum(jy2, y2v[sl])
                                - jnp.maximum(jy1, y1v[sl]), 0.0)
                            inter = iw * ih
                            sup = inter > NMS_THRESH * (ja + areav[sl] - inter)
                            own = jnp.where(sup, keepv[sl], 0.0)
                            own = jnp.where(lane < j, own, 0.0)
                            acc = jnp.maximum(acc, own)
                        anyk = _vmax(acc)
                        onehot = jnp.where(lane == j, 1.0, 0.0)
                        keepv[sl] = keepv[sl] * (1.0 - onehot * anyk)
            return 0
        lax.fori_loop(0, NG, _vreg, 0)

        pltpu.sync_copy(keepv, keep_out)


@jax.jit
def _nms_keep(x1, y1, x2, y2):
    mesh_a = plsc.VectorSubcoreMesh(
        core_axis_name="c", subcore_axis_name="s", num_cores=2)
    overlap = pl.kernel(
        _overlap_body,
        out_type=jax.ShapeDtypeStruct((NPAD,), jnp.float32),
        mesh=mesh_a,
        scratch_types=[
            pltpu.VMEM((NPAD,), jnp.float32),
            pltpu.VMEM((NPAD,), jnp.float32),
            pltpu.VMEM((NPAD,), jnp.float32),
            pltpu.VMEM((NPAD,), jnp.float32),
            pltpu.VMEM((NPAD,), jnp.float32),
            pltpu.VMEM((HB,), jnp.float32),
        ],
    )
    supany = overlap(x1, y1, x2, y2)

    mesh_b = plsc.VectorSubcoreMesh(
        core_axis_name="c", subcore_axis_name="s", num_cores=1)
    resolve = pl.kernel(
        _resolve_body,
        out_type=jax.ShapeDtypeStruct((NPAD,), jnp.float32),
        mesh=mesh_b,
        scratch_types=[
            pltpu.VMEM((NPAD,), jnp.float32),
            pltpu.VMEM((NPAD,), jnp.float32),
            pltpu.VMEM((NPAD,), jnp.float32),
            pltpu.VMEM((NPAD,), jnp.float32),
            pltpu.VMEM((NPAD,), jnp.float32),
            pltpu.VMEM((NPAD,), jnp.float32),
            pltpu.VMEM((NPAD,), jnp.float32),
        ],
    )
    return resolve(x1, y1, x2, y2, supany)


def kernel(boxes, scores, classes):
    max_coord = jnp.max(boxes) + 1.0
    offsets = classes.astype(boxes.dtype) * max_coord
    boxes_off = boxes + offsets[:, None]
    order = jnp.argsort(-scores)
    b_sorted = jnp.take(boxes_off, order, axis=0)
    b_orig_sorted = jnp.take(boxes, order, axis=0)
    s_sorted = jnp.take(scores, order, axis=0)

    pad = jnp.full((NPAD - N,), -1e6, jnp.float32)
    x1 = jnp.concatenate([b_sorted[:, 0], pad])
    y1 = jnp.concatenate([b_sorted[:, 1], pad])
    x2 = jnp.concatenate([b_sorted[:, 2], pad])
    y2 = jnp.concatenate([b_sorted[:, 3], pad])

    keepf = _nms_keep(x1, y1, x2, y2)[:N]
    out = jnp.concatenate(
        [b_orig_sorted * keepf[:, None], (s_sorted * keepf)[:, None]], axis=1)
    return out


# speedup vs baseline: 3.4893x; 1.0772x over previous
import jax
import jax.numpy as jnp
from jax import lax
from jax.experimental import pallas as pl
from jax.experimental.pallas import tpu as pltpu
from jax.experimental.pallas import tpu_sc as plsc

N = 5000
NPAD = 5120
L = 16
NG = NPAD // L
HV = 5
HB = HV * L
NJ = NPAD // HB
NMS_THRESH = 0.75


def _stage_coords(x1h, y1h, x2h, y2h, x1v, y1v, x2v, y2v, areav):
    pltpu.sync_copy(x1h, x1v)
    pltpu.sync_copy(y1h, y1v)
    pltpu.sync_copy(x2h, x2v)
    pltpu.sync_copy(y2h, y2v)

    def _init(i, _):
        sl = pl.ds(i * L, L)
        areav[sl] = (x2v[sl] - x1v[sl]) * (y2v[sl] - y1v[sl])
        return 0
    lax.fori_loop(0, NG, _init, 0)


def _overlap_body(x1h, y1h, x2h, y2h, sup_out,
                  x1v, y1v, x2v, y2v, areav, stage):
    ci = lax.axis_index("c")
    si = lax.axis_index("s")
    wid = ci * 16 + si

    _stage_coords(x1h, y1h, x2h, y2h, x1v, y1v, x2v, y2v, areav)

    lane = lax.iota(jnp.int32, L)
    onehots = [jnp.where(lane == i, 1.0, 0.0) for i in range(L)]

    def _halfjob(m):
        h0 = m * HV
        base = h0 * L
        sls = [pl.ds(base + k * L, L) for k in range(HV)]
        cx1 = [x1v[s] for s in sls]
        cy1 = [y1v[s] for s in sls]
        cx2 = [x2v[s] for s in sls]
        cy2 = [y2v[s] for s in sls]
        ca = [areav[s] for s in sls]

        def _margin(px1, py1, px2, py2, pa, k):
            iw = jnp.maximum(
                jnp.minimum(px2, cx2[k]) - jnp.maximum(px1, cx1[k]), 0.0)
            ih = jnp.maximum(
                jnp.minimum(py2, cy2[k]) - jnp.maximum(py1, cy1[k]), 0.0)
            inter = iw * ih
            return inter - NMS_THRESH * (pa + ca[k] - inter)

        def _pg(g, accs):
            psl = pl.ds(g * L, L)
            px1v, py1v, px2v, py2v, pav = (x1v[psl], y1v[psl], x2v[psl],
                                           y2v[psl], areav[psl])
            accs = list(accs)
            for i in range(L):
                px1, py1, px2, py2, pa = (px1v[i], py1v[i], px2v[i],
                                          py2v[i], pav[i])
                for k in range(HV):
                    accs[k] = jnp.maximum(
                        accs[k], _margin(px1, py1, px2, py2, pa, k))
            return tuple(accs)

        def _pg_self(g, accs):
            psl = pl.ds(g * L, L)
            px1v, py1v, px2v, py2v, pav = (x1v[psl], y1v[psl], x2v[psl],
                                           y2v[psl], areav[psl])
            accs = list(accs)

            def _eq(k):
                d = (g - (h0 + k)).astype(jnp.float32)
                return jnp.maximum(1.0 - d * d, 0.0)
            eqf = [_eq(k) * 1e30 for k in range(HV)]
            for i in range(L):
                px1, py1, px2, py2, pa = (px1v[i], py1v[i], px2v[i],
                                          py2v[i], pav[i])
                for k in range(HV):
                    d = _margin(px1, py1, px2, py2, pa, k)
                    accs[k] = jnp.maximum(accs[k], d - eqf[k] * onehots[i])
            return tuple(accs)

        zero = jnp.zeros((L,), jnp.float32)
        accs = lax.fori_loop(0, h0, _pg, (zero,) * HV)
        accs = lax.fori_loop(h0, h0 + HV, _pg_self, accs)
        for k in range(HV):
            stage[pl.ds(k * L, L)] = accs[k]
        pltpu.sync_copy(stage, sup_out.at[pl.ds(base, HB)])

    def _jobs(t, _):
        _halfjob(wid + t * ((NJ - 1) - 2 * wid))
        return 0
    lax.fori_loop(0, 2, _jobs, 0)


def _resolve_body(x1h, y1h, x2h, y2h, suph, keep_out,
                  x1v, y1v, x2v, y2v, areav, keepv, supv):
    si = lax.axis_index("s")

    @pl.when(si == 0)
    def _():
        _stage_coords(x1h, y1h, x2h, y2h, x1v, y1v, x2v, y2v, areav)
        pltpu.sync_copy(suph, supv)

        def _init(i, _):
            keepv[pl.ds(i * L, L)] = jnp.full((L,), 1.0, jnp.float32)
            return 0
        lax.fori_loop(0, NG, _init, 0)

        lane = lax.iota(jnp.int32, L)

        def _vmax(x):
            vals = [x[i] for i in range(L)]
            while len(vals) > 1:
                vals = [jnp.maximum(vals[i], vals[i + 1])
                        for i in range(0, len(vals), 2)]
            return vals[0]

        def _vreg(v, _):
            sl = pl.ds(v * L, L)
            sv = supv[sl]

            @pl.when(_vmax(sv) > 0.0)
            def _():
                for j in range(L):

                    @pl.when(sv[j] > 0.0)
                    def _():
                        jx1 = x1v[sl][j]
                        jy1 = y1v[sl][j]
                        jx2 = x2v[sl][j]
                        jy2 = y2v[sl][j]
                        ja = areav[sl][j]

                        def _pred(u, acc):
                            ps = pl.ds(u * L, L)
                            iw = jnp.maximum(
                                jnp.minimum(jx2, x2v[ps])
                                - jnp.maximum(jx1, x1v[ps]), 0.0)
                            ih = jnp.maximum(
                                jnp.minimum(jy2, y2v[ps])
                                - jnp.maximum(jy1, y1v[ps]), 0.0)
                            inter = iw * ih
                            sup = inter > NMS_THRESH * (ja + areav[ps] - inter)
                            return jnp.maximum(
                                acc, jnp.where(sup, keepv[ps], 0.0))

                        acc = lax.fori_loop(0, v, _pred,
                                            jnp.zeros((L,), jnp.float32))
                        if j > 0:
                            iw = jnp.maximum(
                                jnp.minimum(jx2, x2v[sl])
                                - jnp.maximum(jx1, x1v[sl]), 0.0)
                            ih = jnp.maximum(
                                jnp.minimum(jy2, y2v[sl])
                                - jnp.maximum(jy1, y1v[sl]), 0.0)
                            inter = iw * ih
                            sup = inter > NMS_THRESH * (ja + areav[sl] - inter)
                            own = jnp.where(sup, keepv[sl], 0.0)
                            own = jnp.where(lane < j, own, 0.0)
                            acc = jnp.maximum(acc, own)
                        anyk = _vmax(acc)
                        onehot = jnp.where(lane == j, 1.0, 0.0)
                        keepv[sl] = keepv[sl] * (1.0 - onehot * anyk)
            return 0
        lax.fori_loop(0, NG, _vreg, 0)

        pltpu.sync_copy(keepv, keep_out)


@jax.jit
def _nms_keep(x1, y1, x2, y2):
    mesh_a = plsc.VectorSubcoreMesh(
        core_axis_name="c", subcore_axis_name="s", num_cores=2)
    overlap = pl.kernel(
        _overlap_body,
        out_type=jax.ShapeDtypeStruct((NPAD,), jnp.float32),
        mesh=mesh_a,
        scratch_types=[
            pltpu.VMEM((NPAD,), jnp.float32),
            pltpu.VMEM((NPAD,), jnp.float32),
            pltpu.VMEM((NPAD,), jnp.float32),
            pltpu.VMEM((NPAD,), jnp.float32),
            pltpu.VMEM((NPAD,), jnp.float32),
            pltpu.VMEM((HB,), jnp.float32),
        ],
    )
    supany = overlap(x1, y1, x2, y2)

    mesh_b = plsc.VectorSubcoreMesh(
        core_axis_name="c", subcore_axis_name="s", num_cores=1)
    resolve = pl.kernel(
        _resolve_body,
        out_type=jax.ShapeDtypeStruct((NPAD,), jnp.float32),
        mesh=mesh_b,
        scratch_types=[
            pltpu.VMEM((NPAD,), jnp.float32),
            pltpu.VMEM((NPAD,), jnp.float32),
            pltpu.VMEM((NPAD,), jnp.float32),
            pltpu.VMEM((NPAD,), jnp.float32),
            pltpu.VMEM((NPAD,), jnp.float32),
            pltpu.VMEM((NPAD,), jnp.float32),
            pltpu.VMEM((NPAD,), jnp.float32),
        ],
    )
    return resolve(x1, y1, x2, y2, supany)


def kernel(boxes, scores, classes):
    max_coord = jnp.max(boxes) + 1.0
    offsets = classes.astype(boxes.dtype) * max_coord
    boxes_off = boxes + offsets[:, None]
    order = jnp.argsort(-scores)
    b_sorted = jnp.take(boxes_off, order, axis=0)
    b_orig_sorted = jnp.take(boxes, order, axis=0)
    s_sorted = jnp.take(scores, order, axis=0)

    pad = jnp.full((NPAD - N,), -1e6, jnp.float32)
    x1 = jnp.concatenate([b_sorted[:, 0], pad])
    y1 = jnp.concatenate([b_sorted[:, 1], pad])
    x2 = jnp.concatenate([b_sorted[:, 2], pad])
    y2 = jnp.concatenate([b_sorted[:, 3], pad])

    keepf = _nms_keep(x1, y1, x2, y2)[:N]
    out = jnp.concatenate(
        [b_orig_sorted * keepf[:, None], (s_sorted * keepf)[:, None]], axis=1)
    return out
